# baseline (device time: 10653 ns/iter reference)
import jax
import jax.numpy as jnp
from jax import lax
from jax.experimental import pallas as pl
from jax.experimental.pallas import tpu as pltpu


def kernel(x, dest):
    n, d = x.shape

    def body(x_ref, dest_ref, out_ref, send_buf, staging, send_sem, recv_sem):
        my_x = lax.axis_index("x")
        my_y = lax.axis_index("y")
        my_z = lax.axis_index("z")
        peer = (my_x, my_y, 1 - my_z)

        barrier_sem = pltpu.get_barrier_semaphore()
        pl.semaphore_signal(
            barrier_sem, inc=1, device_id=peer,
            device_id_type=pl.DeviceIdType.MESH,
        )
        pl.semaphore_wait(barrier_sem, 1)

        dst = dest_ref[:]
        keep = (dst == my_z)
        keep_b = keep.astype(jnp.bfloat16)
        send_b = 1.0 - keep_b

        ri = lax.broadcasted_iota(jnp.int32, (n, n), 0)
        ci = lax.broadcasted_iota(jnp.int32, (n, n), 1)

        upper = (ri < ci).astype(jnp.bfloat16)
        kb = jnp.dot(keep_b, upper, preferred_element_type=jnp.float32)
        kb = kb.astype(jnp.int32)
        col = lax.broadcasted_iota(jnp.int32, (1, n), 1)
        sb = col - kb

        c_keep = jnp.sum(keep_b.astype(jnp.int32))
        c_recv = n - c_keep
        off_keep = my_z * (n - c_keep)
        off_recv = (1 - my_z) * c_keep

        x_b = x_ref[:].astype(jnp.bfloat16)

        p_send = ((ri == sb) & (dst != my_z)).astype(jnp.bfloat16)
        send_buf[:, :] = jnp.dot(
            p_send, x_b, preferred_element_type=jnp.float32
        ).astype(jnp.bfloat16)

        rdma = pltpu.make_async_remote_copy(
            src_ref=send_buf,
            dst_ref=staging,
            send_sem=send_sem,
            recv_sem=recv_sem,
            device_id=peer,
            device_id_type=pl.DeviceIdType.MESH,
        )
        rdma.start()

        p_local = ((ri == off_keep + kb) & keep).astype(jnp.bfloat16)
        local_part = jnp.dot(p_local, x_b, preferred_element_type=jnp.float32)

        rdma.wait()

        p_remote = ((ri == ci + off_recv) & (ci < c_recv)).astype(jnp.bfloat16)
        remote_part = jnp.dot(
            p_remote, staging[:, :], preferred_element_type=jnp.float32
        )
        out_ref[:, :] = local_part + remote_part

    return pl.pallas_call(
        body,
        out_shape=jax.ShapeDtypeStruct((n, d), jnp.float32),
        in_specs=[
            pl.BlockSpec(memory_space=pltpu.VMEM),
            pl.BlockSpec(memory_space=pltpu.VMEM),
        ],
        out_specs=pl.BlockSpec(memory_space=pltpu.VMEM),
        scratch_shapes=[
            pltpu.VMEM((n, d), jnp.bfloat16),
            pltpu.VMEM((n, d), jnp.bfloat16),
            pltpu.SemaphoreType.DMA,
            pltpu.SemaphoreType.DMA,
        ],
        compiler_params=pltpu.CompilerParams(collective_id=0),
    )(x, dest.reshape(1, n))


# device time: 9084 ns/iter; 1.1727x vs baseline; 1.1727x over previous
import jax
import jax.numpy as jnp
from jax import lax
from jax.experimental import pallas as pl
from jax.experimental.pallas import tpu as pltpu

CH = 64


def kernel(x, dest):
    n, d = x.shape
    nc_max = n // CH

    def body(x_ref, dest_ref, out_ref, send_buf, staging, send_sems, recv_sems):
        my_x = lax.axis_index("x")
        my_y = lax.axis_index("y")
        my_z = lax.axis_index("z")
        peer = (my_x, my_y, 1 - my_z)

        barrier_sem = pltpu.get_barrier_semaphore()
        pl.semaphore_signal(
            barrier_sem, inc=1, device_id=peer,
            device_id_type=pl.DeviceIdType.MESH,
        )

        dst = dest_ref[:]
        keep = (dst == my_z).astype(jnp.int32)

        ri = lax.broadcasted_iota(jnp.int32, (n, n), 0)
        ci = lax.broadcasted_iota(jnp.int32, (n, n), 1)

        upper = (ri < ci).astype(jnp.bfloat16)
        kb = jnp.dot(
            keep.astype(jnp.bfloat16), upper,
            preferred_element_type=jnp.float32,
        ).astype(jnp.int32)
        col = lax.broadcasted_iota(jnp.int32, (1, n), 1)
        sb = col - kb

        c_keep = jnp.sum(keep)
        c_send = n - c_keep
        off_keep = my_z * c_send
        off_recv = (1 - my_z) * c_keep
        nchunks = (c_send + CH - 1) // CH

        x_b = x_ref[:].astype(jnp.bfloat16)

        t_send = sb - keep * 2048
        p_send = (ri == t_send).astype(jnp.bfloat16)
        send_buf[:, :] = jnp.dot(
            p_send, x_b, preferred_element_type=jnp.float32
        ).astype(jnp.bfloat16)

        pl.semaphore_wait(barrier_sem, 1)

        def chunk_rdma(k):
            return pltpu.make_async_remote_copy(
                src_ref=send_buf.at[pl.ds(k * CH, CH)],
                dst_ref=staging.at[pl.ds(k * CH, CH)],
                send_sem=send_sems.at[k],
                recv_sem=recv_sems.at[k],
                device_id=peer,
                device_id_type=pl.DeviceIdType.MESH,
            )

        for k in range(nc_max):
            @pl.when(k < nchunks)
            def _(k=k):
                chunk_rdma(k).start()

        t_keep = off_keep + kb - (1 - keep) * 2048
        p_local = (ri == t_keep).astype(jnp.bfloat16)
        local_part = jnp.dot(
            p_local, x_b, preferred_element_type=jnp.float32
        )

        for k in range(nc_max):
            @pl.when(k < nchunks)
            def _(k=k):
                chunk_rdma(k).wait_recv()

        t_recv = off_recv + col + (col >= c_send) * 4096
        p_recv = (ri == t_recv).astype(jnp.bfloat16)
        r1 = lax.broadcasted_iota(jnp.int32, (n, 1), 0)
        staged = jnp.where(r1 < c_send, staging[:, :], jnp.bfloat16(0))
        remote_part = jnp.dot(
            p_recv, staged, preferred_element_type=jnp.float32
        )
        out_ref[:, :] = local_part + remote_part

        for k in range(nc_max):
            @pl.when(k < nchunks)
            def _(k=k):
                chunk_rdma(k).wait_send()

    return pl.pallas_call(
        body,
        out_shape=jax.ShapeDtypeStruct((n, d), jnp.float32),
        in_specs=[
            pl.BlockSpec(memory_space=pltpu.VMEM),
            pl.BlockSpec(memory_space=pltpu.VMEM),
        ],
        out_specs=pl.BlockSpec(memory_space=pltpu.VMEM),
        scratch_shapes=[
            pltpu.VMEM((n, d), jnp.bfloat16),
            pltpu.VMEM((n, d), jnp.bfloat16),
            pltpu.SemaphoreType.DMA((nc_max,)),
            pltpu.SemaphoreType.DMA((nc_max,)),
        ],
        compiler_params=pltpu.CompilerParams(collective_id=0),
    )(x, dest.reshape(1, n))


# device time: 8306 ns/iter; 1.2826x vs baseline; 1.0937x over previous
import jax
import jax.numpy as jnp
from jax import lax
from jax.experimental import pallas as pl
from jax.experimental.pallas import tpu as pltpu

CH = 64


def kernel(x, dest):
    n, d = x.shape
    nc_max = n // CH

    def body(x_ref, dest_ref, tri_ref, out_ref, send_buf, staging, send_sems,
             recv_sems):
        my_x = lax.axis_index("x")
        my_y = lax.axis_index("y")
        my_z = lax.axis_index("z")
        peer = (my_x, my_y, 1 - my_z)

        barrier_sem = pltpu.get_barrier_semaphore()
        pl.semaphore_signal(
            barrier_sem, inc=1, device_id=peer,
            device_id_type=pl.DeviceIdType.MESH,
        )

        dst = dest_ref[:]
        keep = (dst == my_z).astype(jnp.int32)
        kb = jnp.dot(
            keep.astype(jnp.bfloat16), tri_ref[:],
            preferred_element_type=jnp.float32,
        ).astype(jnp.int32)
        col = lax.broadcasted_iota(jnp.int32, (1, n), 1)
        sb = col - kb

        c_keep = jnp.sum(keep)
        c_send = n - c_keep
        off_keep = my_z * c_send
        off_xfer = my_z * c_keep
        off_recv = (1 - my_z) * c_keep
        xfer_end = off_xfer + c_send

        x_b = x_ref[:].astype(jnp.bfloat16)

        t_send = off_xfer + sb - keep * 2048

        pl.semaphore_wait(barrier_sem, 1)

        def chunk_rdma(k):
            return pltpu.make_async_remote_copy(
                src_ref=send_buf.at[pl.ds(k * CH, CH)],
                dst_ref=staging.at[pl.ds(k * CH, CH)],
                send_sem=send_sems.at[k],
                recv_sem=recv_sems.at[k],
                device_id=peer,
                device_id_type=pl.DeviceIdType.MESH,
            )

        def live(k):
            return ((k + 1) * CH > off_xfer) & (k * CH < xfer_end)

        def live_recv(k):
            return ((k + 1) * CH > off_recv) & (k * CH < off_recv + c_send)

        for k in range(nc_max):
            @pl.when(live(k))
            def _(k=k):
                ri_t = k * CH + lax.broadcasted_iota(jnp.int32, (CH, n), 0)
                p_t = (ri_t == t_send).astype(jnp.bfloat16)
                send_buf[pl.ds(k * CH, CH), :] = jnp.dot(
                    p_t, x_b, preferred_element_type=jnp.float32
                ).astype(jnp.bfloat16)
                chunk_rdma(k).start()

        ri = lax.broadcasted_iota(jnp.int32, (n, n), 0)
        t_keep = off_keep + kb - (1 - keep) * 2048
        p_local = (ri == t_keep).astype(jnp.bfloat16)
        local_part = jnp.dot(
            p_local, x_b, preferred_element_type=jnp.float32
        ).astype(jnp.bfloat16)
        r1 = lax.broadcasted_iota(jnp.int32, (n, 1), 0)
        in_recv = (r1 >= off_recv) & (r1 < off_recv + c_send)

        for k in range(nc_max):
            @pl.when(live_recv(k))
            def _(k=k):
                chunk_rdma(k).wait_recv()

        out_ref[:, :] = jnp.where(in_recv, staging[:, :], local_part)

        for k in range(nc_max):
            @pl.when(live(k))
            def _(k=k):
                chunk_rdma(k).wait_send()

    return pl.pallas_call(
        body,
        out_shape=jax.ShapeDtypeStruct((n, d), jnp.bfloat16),
        in_specs=[
            pl.BlockSpec(memory_space=pltpu.VMEM),
            pl.BlockSpec(memory_space=pltpu.VMEM),
            pl.BlockSpec(memory_space=pltpu.VMEM),
        ],
        out_specs=pl.BlockSpec(memory_space=pltpu.VMEM),
        scratch_shapes=[
            pltpu.VMEM((n, d), jnp.bfloat16),
            pltpu.VMEM((n, d), jnp.bfloat16),
            pltpu.SemaphoreType.DMA((nc_max,)),
            pltpu.SemaphoreType.DMA((nc_max,)),
        ],
        compiler_params=pltpu.CompilerParams(collective_id=0),
    )(x, dest.reshape(1, n), jnp.triu(jnp.ones((n, n), jnp.bfloat16), 1))
